# baseline (speedup 1.0000x reference)
import functools

import jax
import jax.numpy as jnp
from jax import lax
from jax.experimental import pallas as pl
from jax.experimental.pallas import tpu as pltpu
from jax.experimental.pallas import tpu_sc as plsc

B, S, H, D = 16, 4096, 16, 64
ROW = S * H * D
NC, NS = 2, 16
NW = NC * NS
PER_ROW = NW // B
CHUNK = ROW // PER_ROW


def _body(cache_h, update_h, seq_h, out_h, sid_v):
    c = lax.axis_index("c")
    s = lax.axis_index("s")
    wid = s * NC + c
    b = wid // PER_ROW
    h = wid % PER_ROW
    pltpu.sync_copy(seq_h, sid_v.at[pl.ds(0, 1)])
    sid = sid_v[...][0]
    off = h * CHUNK

    @pl.when(b != sid)
    def _copy_cache():
        pltpu.sync_copy(cache_h.at[b, pl.ds(off, CHUNK)],
                        out_h.at[b, pl.ds(off, CHUNK)])

    @pl.when(b == sid)
    def _copy_update():
        pltpu.sync_copy(update_h.at[pl.ds(off, CHUNK)],
                        out_h.at[b, pl.ds(off, CHUNK)])


@jax.jit
def _dus(cache2d, update1d, seq_ids):
    mesh = plsc.VectorSubcoreMesh(core_axis_name="c", subcore_axis_name="s")
    k = functools.partial(
        pl.kernel,
        out_type=jax.ShapeDtypeStruct((B, ROW), jnp.float32),
        mesh=mesh,
        scratch_types=[pltpu.VMEM((16,), jnp.int32)],
    )(_body)
    return k(cache2d, update1d, seq_ids)


def kernel(cache, update, seq_ids):
    cache2d = cache.reshape(B, ROW)
    update1d = update.reshape(ROW)
    out = _dus(cache2d, update1d, seq_ids)
    return out.reshape(B, S, H, D)


# speedup vs baseline: 1.0013x; 1.0013x over previous
import jax
import jax.numpy as jnp
from jax.experimental import pallas as pl
from jax.experimental.pallas import tpu as pltpu

B, S, H, D = 16, 4096, 16, 64
ROW = S * H * D


def _body(seq_smem, cache_h, update_h, out_h, sem):
    sid = seq_smem[0]
    for b in range(B):
        @pl.when(b != sid)
        def _copy_row():
            pltpu.make_async_copy(
                cache_h.at[b], out_h.at[b], sem.at[b]).start()

    pltpu.make_async_copy(update_h.at[0], out_h.at[sid], sem.at[B]).start()

    for b in range(B):
        @pl.when(b != sid)
        def _wait_row():
            pltpu.make_async_copy(
                cache_h.at[b], out_h.at[b], sem.at[b]).wait()

    pltpu.make_async_copy(update_h.at[0], out_h.at[sid], sem.at[B]).wait()


@jax.jit
def _dus(cache2d, update2d, seq_ids):
    return pl.pallas_call(
        _body,
        grid_spec=pltpu.PrefetchScalarGridSpec(
            num_scalar_prefetch=1,
            grid=(),
            in_specs=[
                pl.BlockSpec(memory_space=pl.MemorySpace.ANY),
                pl.BlockSpec(memory_space=pl.MemorySpace.ANY),
            ],
            out_specs=pl.BlockSpec(memory_space=pl.MemorySpace.ANY),
            scratch_shapes=[pltpu.SemaphoreType.DMA((B + 1,))],
        ),
        out_shape=jax.ShapeDtypeStruct((B, ROW), jnp.float32),
    )(seq_ids, cache2d, update2d)


def kernel(cache, update, seq_ids):
    cache2d = cache.reshape(B, ROW)
    update2d = update.reshape(1, ROW)
    out = _dus(cache2d, update2d, seq_ids)
    return out.reshape(B, S, H, D)


# speedup vs baseline: 25.8915x; 25.8566x over previous
import jax
import jax.numpy as jnp
from jax.experimental import pallas as pl
from jax.experimental.pallas import tpu as pltpu

B, S, H, D = 16, 4096, 16, 64
HD = H * D
S_BLK = 256
S_BLOCKS = S // S_BLK


def _body(seq_smem, cache_ref, update_ref, out_ref):
    sid = seq_smem[0]
    b = pl.program_id(1)

    @pl.when(b == sid)
    def _from_update():
        out_ref[...] = update_ref[...]

    @pl.when(b != sid)
    def _from_cache():
        out_ref[...] = cache_ref[...]


def _cache_map(s, b, seq):
    sid = seq[0]
    alt = jnp.where(sid == 0, 1, sid - 1)
    return (jnp.where(b == sid, alt, b), s, 0)


def _update_map(s, b, seq):
    return (0, s, 0)


def _out_map(s, b, seq):
    return (b, s, 0)


@jax.jit
def _dus(cache3d, update3d, seq_ids):
    return pl.pallas_call(
        _body,
        grid_spec=pltpu.PrefetchScalarGridSpec(
            num_scalar_prefetch=1,
            grid=(S_BLOCKS, B),
            in_specs=[
                pl.BlockSpec((1, S_BLK, HD), _cache_map),
                pl.BlockSpec((1, S_BLK, HD), _update_map),
            ],
            out_specs=pl.BlockSpec((1, S_BLK, HD), _out_map),
        ),
        out_shape=jax.ShapeDtypeStruct((B, S, HD), jnp.float32),
    )(seq_ids, cache3d, update3d)


def kernel(cache, update, seq_ids):
    cache3d = cache.reshape(B, S, HD)
    update3d = update.reshape(1, S, HD)
    out = _dus(cache3d, update3d, seq_ids)
    return out.reshape(B, S, H, D)


# speedup vs baseline: 29.5772x; 1.1424x over previous
import jax
import jax.numpy as jnp
from jax.experimental import pallas as pl
from jax.experimental.pallas import tpu as pltpu

B, S, H, D = 16, 4096, 16, 64
HD = H * D
S_BLK = 1024
S_BLOCKS = S // S_BLK


def _body(seq_smem, cache_ref, update_ref, out_ref):
    sid = seq_smem[0]
    b = pl.program_id(1)

    @pl.when(b == sid)
    def _from_update():
        out_ref[...] = update_ref[...]

    @pl.when(b != sid)
    def _from_cache():
        out_ref[...] = cache_ref[...]


def _cache_map(s, b, seq):
    sid = seq[0]
    alt = jnp.where(sid == 0, 1, sid - 1)
    return (jnp.where(b == sid, alt, b), s, 0)


def _update_map(s, b, seq):
    return (0, s, 0)


def _out_map(s, b, seq):
    return (b, s, 0)


@jax.jit
def _dus(cache3d, update3d, seq_ids):
    return pl.pallas_call(
        _body,
        grid_spec=pltpu.PrefetchScalarGridSpec(
            num_scalar_prefetch=1,
            grid=(S_BLOCKS, B),
            in_specs=[
                pl.BlockSpec((1, S_BLK, HD), _cache_map),
                pl.BlockSpec((1, S_BLK, HD), _update_map),
            ],
            out_specs=pl.BlockSpec((1, S_BLK, HD), _out_map),
        ),
        out_shape=jax.ShapeDtypeStruct((B, S, HD), jnp.float32),
    )(seq_ids, cache3d, update3d)


def kernel(cache, update, seq_ids):
    cache3d = cache.reshape(B, S, HD)
    update3d = update.reshape(1, S, HD)
    out = _dus(cache3d, update3d, seq_ids)
    return out.reshape(B, S, H, D)
